# Initial kernel scaffold; baseline (speedup 1.0000x reference)
import jax
import jax.numpy as jnp
from jax.experimental import pallas as pl


def kernel(x, edge_index, edge_weight, W1, b1, W2, b2):
    raise NotImplementedError("write your pallas kernel here")


# speedup vs baseline: 28.9979x; 28.9979x over previous
import functools

import jax
import jax.numpy as jnp
from jax import lax
from jax.experimental import pallas as pl
from jax.experimental.pallas import tpu as pltpu
from jax.experimental.pallas import tpu_sc as plsc

N = 10000
NP = 10240
DF = 128
DH = 16
NCLS = 40
E = 320000
NCORE = 2
NSUB = 16
NW = NCORE * NSUB
ROWS = 79
EPAD = NW * ROWS * 128

_mesh = plsc.VectorSubcoreMesh(
    core_axis_name="c", subcore_axis_name="s", num_cores=NCORE, num_subcores=NSUB
)


@functools.partial(
    pl.kernel,
    out_type=jax.ShapeDtypeStruct((NCORE, NP), jnp.float32),
    mesh=_mesh,
    scratch_types=[
        pltpu.VMEM((ROWS, 128), jnp.int32),
        pltpu.VMEM((ROWS, 128), jnp.float32),
        pltpu.VMEM((640,), jnp.float32),
        pltpu.VMEM_SHARED((NP,), jnp.float32),
    ],
)
def _deg_kernel(c3, w3, out, cbuf, wbuf, zbuf, acc):
    cid = lax.axis_index("c")
    sid = lax.axis_index("s")
    wid = sid * NCORE + cid
    pltpu.sync_copy(c3.at[wid], cbuf)
    pltpu.sync_copy(w3.at[wid], wbuf)
    zeros16 = jnp.zeros((16,), jnp.float32)
    for i in range(40):
        zbuf[pl.ds(i * 16, 16)] = zeros16
    pltpu.sync_copy(zbuf, acc.at[pl.ds(sid * 640, 640)])
    plsc.subcore_barrier()

    def body(j, carry):
        pltpu.sync_copy(wbuf.at[j], acc.at[cbuf.at[j]], add=True)
        return carry

    lax.fori_loop(0, ROWS, body, None)
    plsc.subcore_barrier()
    pltpu.sync_copy(acc.at[pl.ds(sid * 640, 640)], out.at[cid, pl.ds(sid * 640, 640)])


@functools.partial(
    pl.kernel,
    out_type=jax.ShapeDtypeStruct((NCORE, NP, DH), jnp.float32),
    mesh=_mesh,
    scratch_types=[
        pltpu.VMEM((ROWS, 128), jnp.int32),
        pltpu.VMEM((ROWS, 128), jnp.int32),
        pltpu.VMEM((ROWS, 128), jnp.float32),
        pltpu.VMEM((128, DH), jnp.float32),
        pltpu.VMEM((128, DH), jnp.float32),
        pltpu.VMEM_SHARED((NP, DH), jnp.float32),
    ],
    compiler_params=pltpu.CompilerParams(use_tc_tiling_on_sc=False),
)
def _layer_kernel(r3, c3, w3, hs, out, rbuf, cbuf, wbuf, rows, zrows, acc):
    cid = lax.axis_index("c")
    sid = lax.axis_index("s")
    wid = sid * NCORE + cid
    pltpu.sync_copy(r3.at[wid], rbuf)
    pltpu.sync_copy(c3.at[wid], cbuf)
    pltpu.sync_copy(w3.at[wid], wbuf)
    zeros16 = jnp.zeros((16,), jnp.float32)
    for k in range(128):
        zrows[k, :] = zeros16
    for k in range(5):
        pltpu.sync_copy(zrows, acc.at[pl.ds(sid * 640 + k * 128, 128)])
    plsc.subcore_barrier()

    def body(j, carry):
        pltpu.sync_copy(hs.at[rbuf.at[j]], rows)
        for g in range(8):
            ew16 = wbuf[j, pl.ds(g * 16, 16)]
            for e in range(16):
                k = g * 16 + e
                rows[k, :] = rows[k, :] * ew16[e]
        pltpu.sync_copy(rows, acc.at[cbuf.at[j]], add=True)
        return carry

    lax.fori_loop(0, ROWS, body, None)
    plsc.subcore_barrier()
    pltpu.sync_copy(acc.at[pl.ds(sid * 640, 640)], out.at[cid, pl.ds(sid * 640, 640)])


def _tc1_body(degT_ref, x_ref, w1_ref, hs_ref, dis_ref):
    deg = degT_ref[:, 0:1] + degT_ref[:, 1:2] + 1.0
    dis = jnp.where(deg > 0.0, lax.rsqrt(deg), 0.0)
    h1 = jnp.dot(x_ref[...], w1_ref[...], preferred_element_type=jnp.float32)
    hs_ref[...] = dis * h1
    dis_ref[...] = dis


def _tc2_body(acc_ref, hs_ref, dis_ref, b1_ref, zs_ref):
    a = acc_ref[0] + acc_ref[1]
    m1 = dis_ref[...] * (a + hs_ref[...]) + b1_ref[...]
    zs_ref[...] = dis_ref[...] * jnp.maximum(m1, 0.0)


def _tc3_body(acc_ref, zs_ref, dis_ref, w2_ref, b2_ref, out_ref):
    m2 = dis_ref[...] * (acc_ref[0] + acc_ref[1] + zs_ref[...])
    logits = jnp.dot(m2, w2_ref[...], preferred_element_type=jnp.float32) + b2_ref[...]
    ls = logits - jnp.max(logits, axis=1, keepdims=True)
    out_ref[...] = ls - jnp.log(jnp.sum(jnp.exp(ls), axis=1, keepdims=True))


def kernel(x, edge_index, edge_weight, W1, b1, W2, b2):
    r = edge_index[0].astype(jnp.int32)
    c = edge_index[1].astype(jnp.int32)
    ew = edge_weight.astype(jnp.float32)
    pad = EPAD - E
    r3 = jnp.pad(r, (0, pad)).reshape(NW, ROWS, 128)
    c3 = jnp.pad(c, (0, pad)).reshape(NW, ROWS, 128)
    w3 = jnp.pad(ew, (0, pad)).reshape(NW, ROWS, 128)

    deg_parts = _deg_kernel(c3, w3)
    degT = deg_parts[:, :N].T

    hs1, dis = pl.pallas_call(
        _tc1_body,
        out_shape=[
            jax.ShapeDtypeStruct((N, DH), jnp.float32),
            jax.ShapeDtypeStruct((N, 1), jnp.float32),
        ],
    )(degT, x, W1)

    acc1 = _layer_kernel(r3, c3, w3, hs1)[:, :N, :]
    zs1 = pl.pallas_call(
        _tc2_body, out_shape=jax.ShapeDtypeStruct((N, DH), jnp.float32)
    )(acc1, hs1, dis, b1.reshape(1, DH))

    acc2 = _layer_kernel(r3, c3, w3, zs1)[:, :N, :]
    out = pl.pallas_call(
        _tc3_body, out_shape=jax.ShapeDtypeStruct((N, NCLS), jnp.float32)
    )(acc2, zs1, dis, W2, b2.reshape(1, NCLS))
    return out


# speedup vs baseline: 44.2887x; 1.5273x over previous
import functools

import jax
import jax.numpy as jnp
from jax import lax
from jax.experimental import pallas as pl
from jax.experimental.pallas import tpu as pltpu
from jax.experimental.pallas import tpu_sc as plsc

N = 10000
NP = 10240
DF = 128
DH = 16
NCLS = 40
E = 320000
NCORE = 2
NSUB = 16
NW = NCORE * NSUB
ROWS = 79
EPAD = NW * ROWS * 128

_mesh = plsc.VectorSubcoreMesh(
    core_axis_name="c", subcore_axis_name="s", num_cores=NCORE, num_subcores=NSUB
)


@functools.partial(
    pl.kernel,
    out_type=jax.ShapeDtypeStruct((NCORE, NP), jnp.float32),
    mesh=_mesh,
    scratch_types=[
        pltpu.VMEM((ROWS, 128), jnp.int32),
        pltpu.VMEM((ROWS, 128), jnp.float32),
        pltpu.VMEM((640,), jnp.float32),
        pltpu.VMEM_SHARED((NP,), jnp.float32),
    ],
)
def _deg_kernel(c3, w3, out, cbuf, wbuf, zbuf, acc):
    cid = lax.axis_index("c")
    sid = lax.axis_index("s")
    wid = sid * NCORE + cid
    pltpu.sync_copy(c3.at[wid], cbuf)
    pltpu.sync_copy(w3.at[wid], wbuf)
    zeros16 = jnp.zeros((16,), jnp.float32)
    for i in range(40):
        zbuf[pl.ds(i * 16, 16)] = zeros16
    pltpu.sync_copy(zbuf, acc.at[pl.ds(sid * 640, 640)])
    plsc.subcore_barrier()

    def body(j, carry):
        pltpu.sync_copy(wbuf.at[j], acc.at[cbuf.at[j]], add=True)
        return carry

    lax.fori_loop(0, ROWS, body, None)
    plsc.subcore_barrier()
    pltpu.sync_copy(acc.at[pl.ds(sid * 640, 640)], out.at[cid, pl.ds(sid * 640, 640)])


@functools.partial(
    pl.kernel,
    out_type=jax.ShapeDtypeStruct((NCORE, NP, DH), jnp.float32),
    mesh=_mesh,
    scratch_types=[
        pltpu.VMEM((ROWS, 128), jnp.int32),
        pltpu.VMEM((ROWS, 128), jnp.int32),
        pltpu.VMEM((ROWS, 128), jnp.float32),
        pltpu.VMEM((128, DH), jnp.float32),
        pltpu.VMEM((128, DH), jnp.float32),
        pltpu.VMEM_SHARED((NP, DH), jnp.float32),
        pltpu.VMEM_SHARED((NP, DH), jnp.float32),
    ],
    compiler_params=pltpu.CompilerParams(use_tc_tiling_on_sc=False),
)
def _layer_kernel(r3, c3, w3, hs, out, rbuf, cbuf, wbuf, rows, zrows, acc, hsbuf):
    cid = lax.axis_index("c")
    sid = lax.axis_index("s")
    wid = sid * NCORE + cid
    pltpu.sync_copy(r3.at[wid], rbuf)
    pltpu.sync_copy(c3.at[wid], cbuf)
    pltpu.sync_copy(w3.at[wid], wbuf)
    pltpu.sync_copy(hs.at[pl.ds(sid * 625, 625)], hsbuf.at[pl.ds(sid * 625, 625)])
    zeros16 = jnp.zeros((16,), jnp.float32)
    for k in range(128):
        zrows[k, :] = zeros16
    for k in range(5):
        pltpu.sync_copy(zrows, acc.at[pl.ds(sid * 640 + k * 128, 128)])
    plsc.subcore_barrier()

    def body(j, carry):
        pltpu.sync_copy(hsbuf.at[rbuf.at[j]], rows)
        for g in range(8):
            ew16 = wbuf[j, pl.ds(g * 16, 16)]
            for e in range(16):
                k = g * 16 + e
                rows[k, :] = rows[k, :] * ew16[e]
        pltpu.sync_copy(rows, acc.at[cbuf.at[j]], add=True)
        return carry

    lax.fori_loop(0, ROWS, body, None)
    plsc.subcore_barrier()
    pltpu.sync_copy(acc.at[pl.ds(sid * 640, 640)], out.at[cid, pl.ds(sid * 640, 640)])


def _tc1_body(degT_ref, x_ref, w1_ref, hs_ref, dis_ref):
    deg = degT_ref[:, 0:1] + degT_ref[:, 1:2] + 1.0
    dis = jnp.where(deg > 0.0, lax.rsqrt(deg), 0.0)
    h1 = jnp.dot(x_ref[...], w1_ref[...], preferred_element_type=jnp.float32)
    hs_ref[...] = dis * h1
    dis_ref[...] = dis


def _tc2_body(acc_ref, hs_ref, dis_ref, b1_ref, zs_ref):
    a = acc_ref[0] + acc_ref[1]
    m1 = dis_ref[...] * (a + hs_ref[...]) + b1_ref[...]
    zs_ref[...] = dis_ref[...] * jnp.maximum(m1, 0.0)


def _tc3_body(acc_ref, zs_ref, dis_ref, w2_ref, b2_ref, out_ref):
    m2 = dis_ref[...] * (acc_ref[0] + acc_ref[1] + zs_ref[...])
    logits = jnp.dot(m2, w2_ref[...], preferred_element_type=jnp.float32) + b2_ref[...]
    ls = logits - jnp.max(logits, axis=1, keepdims=True)
    out_ref[...] = ls - jnp.log(jnp.sum(jnp.exp(ls), axis=1, keepdims=True))


def kernel(x, edge_index, edge_weight, W1, b1, W2, b2):
    r = edge_index[0].astype(jnp.int32)
    c = edge_index[1].astype(jnp.int32)
    ew = edge_weight.astype(jnp.float32)
    pad = EPAD - E
    r3 = jnp.pad(r, (0, pad)).reshape(NW, ROWS, 128)
    c3 = jnp.pad(c, (0, pad)).reshape(NW, ROWS, 128)
    w3 = jnp.pad(ew, (0, pad)).reshape(NW, ROWS, 128)

    deg_parts = _deg_kernel(c3, w3)
    degT = deg_parts[:, :N].T

    hs1, dis = pl.pallas_call(
        _tc1_body,
        out_shape=[
            jax.ShapeDtypeStruct((N, DH), jnp.float32),
            jax.ShapeDtypeStruct((N, 1), jnp.float32),
        ],
    )(degT, x, W1)

    acc1 = _layer_kernel(r3, c3, w3, hs1)[:, :N, :]
    zs1 = pl.pallas_call(
        _tc2_body, out_shape=jax.ShapeDtypeStruct((N, DH), jnp.float32)
    )(acc1, hs1, dis, b1.reshape(1, DH))

    acc2 = _layer_kernel(r3, c3, w3, zs1)[:, :N, :]
    out = pl.pallas_call(
        _tc3_body, out_shape=jax.ShapeDtypeStruct((N, NCLS), jnp.float32)
    )(acc2, zs1, dis, W2, b2.reshape(1, NCLS))
    return out


# speedup vs baseline: 49.2570x; 1.1122x over previous
import functools

import jax
import jax.numpy as jnp
from jax import lax
from jax.experimental import pallas as pl
from jax.experimental.pallas import tpu as pltpu
from jax.experimental.pallas import tpu_sc as plsc

N = 10000
NP = 10240
NG = NP // 16
DF = 128
DH = 16
NCLS = 40
E = 320000
NCORE = 2
NSUB = 16
NW = NCORE * NSUB
ROWS = 79
EPAD = NW * ROWS * 128

_mesh = plsc.VectorSubcoreMesh(
    core_axis_name="c", subcore_axis_name="s", num_cores=NCORE, num_subcores=NSUB
)


@functools.partial(
    pl.kernel,
    out_type=jax.ShapeDtypeStruct((NCORE, NP), jnp.float32),
    mesh=_mesh,
    scratch_types=[
        pltpu.VMEM((ROWS, 128), jnp.int32),
        pltpu.VMEM((ROWS, 128), jnp.float32),
        pltpu.VMEM((640,), jnp.float32),
        pltpu.VMEM_SHARED((NP,), jnp.float32),
    ],
)
def _deg_kernel(c3, w3, out, cbuf, wbuf, zbuf, acc):
    cid = lax.axis_index("c")
    sid = lax.axis_index("s")
    wid = sid * NCORE + cid
    pltpu.sync_copy(c3.at[wid], cbuf)
    pltpu.sync_copy(w3.at[wid], wbuf)
    zeros16 = jnp.zeros((16,), jnp.float32)
    for i in range(40):
        zbuf[pl.ds(i * 16, 16)] = zeros16
    pltpu.sync_copy(zbuf, acc.at[pl.ds(sid * 640, 640)])
    plsc.subcore_barrier()

    def body(j, carry):
        pltpu.sync_copy(wbuf.at[j], acc.at[cbuf.at[j]], add=True)
        return carry

    lax.fori_loop(0, ROWS, body, None)
    plsc.subcore_barrier()
    pltpu.sync_copy(acc.at[pl.ds(sid * 640, 640)], out.at[cid, pl.ds(sid * 640, 640)])


@functools.partial(
    pl.kernel,
    out_type=jax.ShapeDtypeStruct((NCORE, NP, DH), jnp.float32),
    mesh=_mesh,
    scratch_types=[
        pltpu.VMEM((ROWS, 128), jnp.int32),
        pltpu.VMEM((ROWS, 128), jnp.int32),
        pltpu.VMEM((ROWS, 128), jnp.float32),
        pltpu.VMEM((128, DH), jnp.float32),
        pltpu.VMEM((128, DH), jnp.float32),
        pltpu.VMEM_SHARED((NP, DH), jnp.float32),
        pltpu.VMEM_SHARED((NP, DH), jnp.float32),
    ],
    compiler_params=pltpu.CompilerParams(use_tc_tiling_on_sc=False),
)
def _layer1_kernel(r3, c3, w3, hs, out, rbuf, cbuf, wbuf, rows, zrows, acc, hsbuf):
    cid = lax.axis_index("c")
    sid = lax.axis_index("s")
    wid = sid * NCORE + cid
    pltpu.sync_copy(r3.at[wid], rbuf)
    pltpu.sync_copy(c3.at[wid], cbuf)
    pltpu.sync_copy(w3.at[wid], wbuf)
    pltpu.sync_copy(hs.at[pl.ds(sid * 640, 640)], hsbuf.at[pl.ds(sid * 640, 640)])
    zeros16 = jnp.zeros((16,), jnp.float32)
    for k in range(128):
        zrows[k, :] = zeros16
    for k in range(5):
        pltpu.sync_copy(zrows, acc.at[pl.ds(sid * 640 + k * 128, 128)])
    plsc.subcore_barrier()

    def body(j, carry):
        pltpu.sync_copy(hsbuf.at[rbuf.at[j]], rows)
        for g in range(8):
            ew16 = wbuf[j, pl.ds(g * 16, 16)]
            for e in range(16):
                k = g * 16 + e
                rows[k, :] = rows[k, :] * ew16[e]
        pltpu.sync_copy(rows, acc.at[cbuf.at[j]], add=True)
        return carry

    lax.fori_loop(0, ROWS, body, None)
    plsc.subcore_barrier()
    pltpu.sync_copy(acc.at[pl.ds(sid * 640, 640)], out.at[cid, pl.ds(sid * 640, 640)])


@functools.partial(
    pl.kernel,
    out_type=[
        jax.ShapeDtypeStruct((NCORE, NP, DH), jnp.float32),
        jax.ShapeDtypeStruct((NP, DH), jnp.float32),
    ],
    mesh=_mesh,
    scratch_types=[
        pltpu.VMEM((ROWS, 128), jnp.int32),
        pltpu.VMEM((ROWS, 128), jnp.int32),
        pltpu.VMEM((ROWS, 128), jnp.float32),
        pltpu.VMEM((128, DH), jnp.float32),
        pltpu.VMEM((128, DH), jnp.float32),
        pltpu.VMEM((640, DH), jnp.float32),
        pltpu.VMEM((640, DH), jnp.float32),
        pltpu.VMEM((640, DH), jnp.float32),
        pltpu.VMEM((40, 16), jnp.float32),
        pltpu.VMEM((16,), jnp.float32),
        pltpu.VMEM((640, DH), jnp.float32),
        pltpu.VMEM_SHARED((NP, DH), jnp.float32),
        pltpu.VMEM_SHARED((NP, DH), jnp.float32),
    ],
    compiler_params=pltpu.CompilerParams(use_tc_tiling_on_sc=False),
)
def _layer2_kernel(r3, c3, w3, accp, hs, dis2d, b1, out, zs_out,
                   rbuf, cbuf, wbuf, rows, zrows, a0, a1, hsv, disv, b1v, zsv,
                   acc, hsbuf):
    cid = lax.axis_index("c")
    sid = lax.axis_index("s")
    wid = sid * NCORE + cid
    pltpu.sync_copy(r3.at[wid], rbuf)
    pltpu.sync_copy(c3.at[wid], cbuf)
    pltpu.sync_copy(w3.at[wid], wbuf)
    base = sid * 640
    pltpu.sync_copy(accp.at[0, pl.ds(base, 640)], a0)
    pltpu.sync_copy(accp.at[1, pl.ds(base, 640)], a1)
    pltpu.sync_copy(hs.at[pl.ds(base, 640)], hsv)
    pltpu.sync_copy(dis2d.at[pl.ds(sid * 40, 40)], disv)
    pltpu.sync_copy(b1, b1v)
    b1r = b1v[...]

    def pro(g, carry):
        dis16 = disv[g]
        for e in range(16):
            k = g * 16 + e
            d = dis16[e]
            t = a0[k, :] + a1[k, :] + hsv[k, :]
            m = t * d + b1r
            zsv[k, :] = jnp.maximum(m, 0.0) * d
        return carry

    lax.fori_loop(0, 40, pro, None)
    pltpu.sync_copy(zsv, hsbuf.at[pl.ds(base, 640)])

    @pl.when(cid == 0)
    def _():
        pltpu.sync_copy(zsv, zs_out.at[pl.ds(base, 640)])

    zeros16 = jnp.zeros((16,), jnp.float32)
    for k in range(128):
        zrows[k, :] = zeros16
    for k in range(5):
        pltpu.sync_copy(zrows, acc.at[pl.ds(base + k * 128, 128)])
    plsc.subcore_barrier()

    def body(j, carry):
        pltpu.sync_copy(hsbuf.at[rbuf.at[j]], rows)
        for g in range(8):
            ew16 = wbuf[j, pl.ds(g * 16, 16)]
            for e in range(16):
                k = g * 16 + e
                rows[k, :] = rows[k, :] * ew16[e]
        pltpu.sync_copy(rows, acc.at[cbuf.at[j]], add=True)
        return carry

    lax.fori_loop(0, ROWS, body, None)
    plsc.subcore_barrier()
    pltpu.sync_copy(acc.at[pl.ds(base, 640)], out.at[cid, pl.ds(base, 640)])


def _tc1_body(degT_ref, degA_ref, degB_ref, x_ref, w1_ref, hs_ref, dis_ref,
              discol_ref):
    deg = degT_ref[:, 0:1] + degT_ref[:, 1:2] + 1.0
    dis = jnp.where(deg > 0.0, lax.rsqrt(deg), 0.0)
    h1 = jnp.dot(x_ref[...], w1_ref[...], preferred_element_type=jnp.float32)
    hs_ref[0:N, :] = dis * h1
    hs_ref[N:NP, :] = jnp.zeros((NP - N, DH), jnp.float32)
    discol_ref[0:N, :] = dis
    discol_ref[N:NP, :] = jnp.zeros((NP - N, 1), jnp.float32)
    deg2 = degA_ref[...] + degB_ref[...] + 1.0
    dis_ref[...] = jnp.where(deg2 > 0.0, lax.rsqrt(deg2), 0.0)


def _tc3_body(acc_ref, zs_ref, dis_ref, w2_ref, b2_ref, out_ref):
    dis = dis_ref[0:N, :]
    a = acc_ref[0, 0:N, :] + acc_ref[1, 0:N, :]
    m2 = dis * (a + zs_ref[0:N, :])
    logits = jnp.dot(m2, w2_ref[...], preferred_element_type=jnp.float32) + b2_ref[...]
    ls = logits - jnp.max(logits, axis=1, keepdims=True)
    out_ref[...] = ls - jnp.log(jnp.sum(jnp.exp(ls), axis=1, keepdims=True))


def kernel(x, edge_index, edge_weight, W1, b1, W2, b2):
    r = edge_index[0].astype(jnp.int32)
    c = edge_index[1].astype(jnp.int32)
    ew = edge_weight.astype(jnp.float32)
    pad = EPAD - E
    r3 = jnp.pad(r, (0, pad)).reshape(NW, ROWS, 128)
    c3 = jnp.pad(c, (0, pad)).reshape(NW, ROWS, 128)
    w3 = jnp.pad(ew, (0, pad)).reshape(NW, ROWS, 128)

    deg_parts = _deg_kernel(c3, w3)
    degT = deg_parts[:, :N].T
    degA = deg_parts[0].reshape(NG, 16)
    degB = deg_parts[1].reshape(NG, 16)

    hs1, dis2d, discol = pl.pallas_call(
        _tc1_body,
        out_shape=[
            jax.ShapeDtypeStruct((NP, DH), jnp.float32),
            jax.ShapeDtypeStruct((NG, 16), jnp.float32),
            jax.ShapeDtypeStruct((NP, 1), jnp.float32),
        ],
    )(degT, degA, degB, x, W1)

    acc1 = _layer1_kernel(r3, c3, w3, hs1)
    acc2, zs1 = _layer2_kernel(r3, c3, w3, acc1, hs1, dis2d, b1)
    out = pl.pallas_call(
        _tc3_body, out_shape=jax.ShapeDtypeStruct((N, NCLS), jnp.float32)
    )(acc2, zs1, discol, W2, b2.reshape(1, NCLS))
    return out


# speedup vs baseline: 57.1611x; 1.1605x over previous
import functools

import jax
import jax.numpy as jnp
from jax import lax
from jax.experimental import pallas as pl
from jax.experimental.pallas import tpu as pltpu
from jax.experimental.pallas import tpu_sc as plsc

N = 10000
NP = 10240
NG = NP // 16
DF = 128
DH = 16
NCLS = 40
E = 320000
NCORE = 2
NSUB = 16
NW = NCORE * NSUB
ROWS = 80
EPAD = NW * ROWS * 128

_mesh = plsc.VectorSubcoreMesh(
    core_axis_name="c", subcore_axis_name="s", num_cores=NCORE, num_subcores=NSUB
)


@functools.partial(
    pl.kernel,
    out_type=jax.ShapeDtypeStruct((NCORE, NP), jnp.float32),
    mesh=_mesh,
    scratch_types=[
        pltpu.VMEM((ROWS, 128), jnp.int32),
        pltpu.VMEM((ROWS, 128), jnp.float32),
        pltpu.VMEM((640,), jnp.float32),
        pltpu.VMEM_SHARED((NP,), jnp.float32),
    ],
)
def _deg_kernel(c3, w3, out, cbuf, wbuf, zbuf, acc):
    cid = lax.axis_index("c")
    sid = lax.axis_index("s")
    wid = sid * NCORE + cid
    pltpu.sync_copy(c3.at[wid], cbuf)
    pltpu.sync_copy(w3.at[wid], wbuf)
    zeros16 = jnp.zeros((16,), jnp.float32)
    for i in range(40):
        zbuf[pl.ds(i * 16, 16)] = zeros16
    pltpu.sync_copy(zbuf, acc.at[pl.ds(sid * 640, 640)])
    plsc.subcore_barrier()

    def body(j, carry):
        pltpu.sync_copy(wbuf.at[j], acc.at[cbuf.at[j]], add=True)
        return carry

    lax.fori_loop(0, ROWS, body, None)
    plsc.subcore_barrier()
    pltpu.sync_copy(acc.at[pl.ds(sid * 640, 640)], out.at[cid, pl.ds(sid * 640, 640)])


@functools.partial(
    pl.kernel,
    out_type=jax.ShapeDtypeStruct((NCORE, NP, DH), jnp.float32),
    mesh=_mesh,
    scratch_types=[
        pltpu.VMEM((ROWS, 128), jnp.int32),
        pltpu.VMEM((ROWS, 128), jnp.int32),
        pltpu.VMEM((ROWS, 128), jnp.float32),
        pltpu.VMEM((128, DH), jnp.float32),
        pltpu.VMEM((128, DH), jnp.float32),
        pltpu.VMEM((128, DH), jnp.float32),
        pltpu.VMEM_SHARED((NP, DH), jnp.float32),
        pltpu.VMEM_SHARED((NP, DH), jnp.float32),
        pltpu.SemaphoreType.DMA,
        pltpu.SemaphoreType.DMA,
    ],
    compiler_params=pltpu.CompilerParams(use_tc_tiling_on_sc=False),
)
def _layer1_kernel(r3, c3, w3, hs, out, rbuf, cbuf, wbuf, rows0, rows1, zrows,
                   acc, hsbuf, sem0, sem1):
    cid = lax.axis_index("c")
    sid = lax.axis_index("s")
    wid = sid * NCORE + cid
    pltpu.sync_copy(r3.at[wid], rbuf)
    pltpu.sync_copy(c3.at[wid], cbuf)
    pltpu.sync_copy(w3.at[wid], wbuf)
    pltpu.sync_copy(hs.at[pl.ds(sid * 640, 640)], hsbuf.at[pl.ds(sid * 640, 640)])
    zeros16 = jnp.zeros((16,), jnp.float32)
    for k in range(128):
        zrows[k, :] = zeros16
    for k in range(5):
        pltpu.sync_copy(zrows, acc.at[pl.ds(sid * 640 + k * 128, 128)])
    plsc.subcore_barrier()

    pltpu.async_copy(hsbuf.at[rbuf.at[0]], rows0, sem0)
    pltpu.async_copy(hsbuf.at[rbuf.at[1]], rows1, sem1)

    def step(j, rows, sem, issue_next):
        pltpu.make_async_copy(hsbuf.at[rbuf.at[j]], rows, sem).wait()
        for g in range(8):
            ew16 = wbuf[j, pl.ds(g * 16, 16)]
            for e in range(16):
                k = g * 16 + e
                rows[k, :] = rows[k, :] * ew16[e]
        pltpu.sync_copy(rows, acc.at[cbuf.at[j]], add=True)
        if issue_next:
            pltpu.async_copy(hsbuf.at[rbuf.at[j + 2]], rows, sem)

    def body(i, carry):
        g = i * 2
        step(g, rows0, sem0, True)
        step(g + 1, rows1, sem1, True)
        return carry

    lax.fori_loop(0, (ROWS - 2) // 2, body, None)
    step(ROWS - 2, rows0, sem0, False)
    step(ROWS - 1, rows1, sem1, False)
    plsc.subcore_barrier()
    pltpu.sync_copy(acc.at[pl.ds(sid * 640, 640)], out.at[cid, pl.ds(sid * 640, 640)])


@functools.partial(
    pl.kernel,
    out_type=[
        jax.ShapeDtypeStruct((NCORE, NP, DH), jnp.float32),
        jax.ShapeDtypeStruct((NP, DH), jnp.float32),
    ],
    mesh=_mesh,
    scratch_types=[
        pltpu.VMEM((ROWS, 128), jnp.int32),
        pltpu.VMEM((ROWS, 128), jnp.int32),
        pltpu.VMEM((ROWS, 128), jnp.float32),
        pltpu.VMEM((128, DH), jnp.float32),
        pltpu.VMEM((128, DH), jnp.float32),
        pltpu.VMEM((128, DH), jnp.float32),
        pltpu.VMEM((640, DH), jnp.float32),
        pltpu.VMEM((640, DH), jnp.float32),
        pltpu.VMEM((640, DH), jnp.float32),
        pltpu.VMEM((40, 16), jnp.float32),
        pltpu.VMEM((16,), jnp.float32),
        pltpu.VMEM((640, DH), jnp.float32),
        pltpu.VMEM_SHARED((NP, DH), jnp.float32),
        pltpu.VMEM_SHARED((NP, DH), jnp.float32),
        pltpu.SemaphoreType.DMA,
        pltpu.SemaphoreType.DMA,
    ],
    compiler_params=pltpu.CompilerParams(use_tc_tiling_on_sc=False),
)
def _layer2_kernel(r3, c3, w3, accp, hs, dis2d, b1, out, zs_out,
                   rbuf, cbuf, wbuf, rows0, rows1, zrows, a0, a1, hsv, disv,
                   b1v, zsv, acc, hsbuf, sem0, sem1):
    cid = lax.axis_index("c")
    sid = lax.axis_index("s")
    wid = sid * NCORE + cid
    pltpu.sync_copy(r3.at[wid], rbuf)
    pltpu.sync_copy(c3.at[wid], cbuf)
    pltpu.sync_copy(w3.at[wid], wbuf)
    base = sid * 640
    pltpu.sync_copy(accp.at[0, pl.ds(base, 640)], a0)
    pltpu.sync_copy(accp.at[1, pl.ds(base, 640)], a1)
    pltpu.sync_copy(hs.at[pl.ds(base, 640)], hsv)
    pltpu.sync_copy(dis2d.at[pl.ds(sid * 40, 40)], disv)
    pltpu.sync_copy(b1, b1v)
    b1r = b1v[...]

    def pro(g, carry):
        dis16 = disv[g]
        for e in range(16):
            k = g * 16 + e
            d = dis16[e]
            t = a0[k, :] + a1[k, :] + hsv[k, :]
            m = t * d + b1r
            zsv[k, :] = jnp.maximum(m, 0.0) * d
        return carry

    lax.fori_loop(0, 40, pro, None)
    pltpu.sync_copy(zsv, hsbuf.at[pl.ds(base, 640)])

    @pl.when(cid == 0)
    def _():
        pltpu.sync_copy(zsv, zs_out.at[pl.ds(base, 640)])

    zeros16 = jnp.zeros((16,), jnp.float32)
    for k in range(128):
        zrows[k, :] = zeros16
    for k in range(5):
        pltpu.sync_copy(zrows, acc.at[pl.ds(base + k * 128, 128)])
    plsc.subcore_barrier()

    pltpu.async_copy(hsbuf.at[rbuf.at[0]], rows0, sem0)
    pltpu.async_copy(hsbuf.at[rbuf.at[1]], rows1, sem1)

    def step(j, rows, sem, issue_next):
        pltpu.make_async_copy(hsbuf.at[rbuf.at[j]], rows, sem).wait()
        for g in range(8):
            ew16 = wbuf[j, pl.ds(g * 16, 16)]
            for e in range(16):
                k = g * 16 + e
                rows[k, :] = rows[k, :] * ew16[e]
        pltpu.sync_copy(rows, acc.at[cbuf.at[j]], add=True)
        if issue_next:
            pltpu.async_copy(hsbuf.at[rbuf.at[j + 2]], rows, sem)

    def body(i, carry):
        g = i * 2
        step(g, rows0, sem0, True)
        step(g + 1, rows1, sem1, True)
        return carry

    lax.fori_loop(0, (ROWS - 2) // 2, body, None)
    step(ROWS - 2, rows0, sem0, False)
    step(ROWS - 1, rows1, sem1, False)
    plsc.subcore_barrier()
    pltpu.sync_copy(acc.at[pl.ds(base, 640)], out.at[cid, pl.ds(base, 640)])


def _tc1_body(degT_ref, degA_ref, degB_ref, x_ref, w1_ref, hs_ref, dis_ref,
              discol_ref):
    deg = degT_ref[:, 0:1] + degT_ref[:, 1:2] + 1.0
    dis = jnp.where(deg > 0.0, lax.rsqrt(deg), 0.0)
    h1 = jnp.dot(x_ref[...], w1_ref[...], preferred_element_type=jnp.float32)
    hs_ref[0:N, :] = dis * h1
    hs_ref[N:NP, :] = jnp.zeros((NP - N, DH), jnp.float32)
    discol_ref[0:N, :] = dis
    discol_ref[N:NP, :] = jnp.zeros((NP - N, 1), jnp.float32)
    deg2 = degA_ref[...] + degB_ref[...] + 1.0
    dis_ref[...] = jnp.where(deg2 > 0.0, lax.rsqrt(deg2), 0.0)


def _tc3_body(acc_ref, zs_ref, dis_ref, w2_ref, b2_ref, out_ref):
    dis = dis_ref[0:N, :]
    a = acc_ref[0, 0:N, :] + acc_ref[1, 0:N, :]
    m2 = dis * (a + zs_ref[0:N, :])
    logits = jnp.dot(m2, w2_ref[...], preferred_element_type=jnp.float32) + b2_ref[...]
    ls = logits - jnp.max(logits, axis=1, keepdims=True)
    out_ref[...] = ls - jnp.log(jnp.sum(jnp.exp(ls), axis=1, keepdims=True))


def kernel(x, edge_index, edge_weight, W1, b1, W2, b2):
    r = edge_index[0].astype(jnp.int32)
    c = edge_index[1].astype(jnp.int32)
    ew = edge_weight.astype(jnp.float32)
    pad = EPAD - E
    r3 = jnp.pad(r, (0, pad)).reshape(NW, ROWS, 128)
    c3 = jnp.pad(c, (0, pad)).reshape(NW, ROWS, 128)
    w3 = jnp.pad(ew, (0, pad)).reshape(NW, ROWS, 128)

    deg_parts = _deg_kernel(c3, w3)
    degT = deg_parts[:, :N].T
    degA = deg_parts[0].reshape(NG, 16)
    degB = deg_parts[1].reshape(NG, 16)

    hs1, dis2d, discol = pl.pallas_call(
        _tc1_body,
        out_shape=[
            jax.ShapeDtypeStruct((NP, DH), jnp.float32),
            jax.ShapeDtypeStruct((NG, 16), jnp.float32),
            jax.ShapeDtypeStruct((NP, 1), jnp.float32),
        ],
    )(degT, degA, degB, x, W1)

    acc1 = _layer1_kernel(r3, c3, w3, hs1)
    acc2, zs1 = _layer2_kernel(r3, c3, w3, acc1, hs1, dis2d, b1)
    out = pl.pallas_call(
        _tc3_body, out_shape=jax.ShapeDtypeStruct((N, NCLS), jnp.float32)
    )(acc2, zs1, discol, W2, b2.reshape(1, NCLS))
    return out
